# Initial kernel scaffold; baseline (speedup 1.0000x reference)
import jax
import jax.numpy as jnp
from jax.experimental import pallas as pl


def kernel(sub, rel, edge_index, edge_type, ent_feature, id_embed, gender_table, age_table, level_table, init_rel, loop_rel, w_in, w_out, w_loop, w_rel, b_conv):
    raise NotImplementedError("write your pallas kernel here")


# speedup vs baseline: 1.0708x; 1.0708x over previous
import functools

import jax
import jax.numpy as jnp
from jax import lax
from jax.experimental import pallas as pl
from jax.experimental.pallas import tpu as pltpu
from jax.experimental.pallas import tpu_sc as plsc

N = 50000
NREL = 41
NT = 20
E = 800000
H = 400000
D = 64
DO = 128
B = 4096

EPD = 409600
PAD = EPD - H
PER_SUB = 25600
K = 1024
CHUNKS = PER_SUB // K
KC = 512
CCHUNKS = PER_SUB // KC
NPAD = 51200
ZROWS = 3200
NTG = 5
NBG = NT // NTG
BFLAT = NPAD * NTG
BSUB = BFLAT // 16
BCH = 3200
BTRASH = BFLAT - 1
NPASS = 7
PROWS = 7680
RR = 7936
RSUB = 496
RZCH = 248
OSUB = 480
OCH = 240
TRASH = 7680

_mesh = plsc.VectorSubcoreMesh(core_axis_name="c", subcore_axis_name="s")


@functools.partial(
    pl.kernel, mesh=_mesh,
    out_type=jax.ShapeDtypeStruct((2 * NPAD,), jnp.float32),
    scratch_types=[
        pltpu.VMEM((K,), jnp.int32),
        pltpu.VMEM((K,), jnp.float32),
        pltpu.VMEM_SHARED((NPAD,), jnp.float32),
    ],
)
def _deg_kernel(rows_hbm, zeros_hbm, deg_hbm, idx_v, ones_v, deg_sp):
    c = lax.axis_index("c")
    s = lax.axis_index("s")
    zoff = pl.multiple_of(s * ZROWS, 8)
    pltpu.sync_copy(zeros_hbm.at[pl.ds(0, ZROWS)], deg_sp.at[pl.ds(zoff, ZROWS)])
    for i in range(K // 16):
        ones_v[pl.ds(i * 16, 16)] = jnp.full((16,), 1.0, jnp.float32)
    plsc.subcore_barrier()

    base = pl.multiple_of(c * EPD + s * PER_SUB, 8)

    def chunk(k, carry):
        off = pl.multiple_of(base + k * K, 8)
        pltpu.sync_copy(rows_hbm.at[pl.ds(off, K)], idx_v)
        pltpu.sync_copy(ones_v, deg_sp.at[idx_v], add=True)
        return carry

    lax.fori_loop(0, CHUNKS, chunk, 0)
    plsc.subcore_barrier()
    ooff = pl.multiple_of(c * NPAD + s * ZROWS, 8)
    pltpu.sync_copy(deg_sp.at[pl.ds(zoff, ZROWS)], deg_hbm.at[pl.ds(ooff, ZROWS)])


@functools.partial(
    pl.kernel, mesh=_mesh,
    out_type=jax.ShapeDtypeStruct((2 * NPASS * PROWS, DO), jnp.float32),
    scratch_types=[
        pltpu.VMEM((KC,), jnp.int32),
        pltpu.VMEM((KC,), jnp.int32),
        pltpu.VMEM((KC, DO), jnp.float32),
        pltpu.VMEM_SHARED((RR, DO), jnp.float32),
        pltpu.SemaphoreType.DMA,
    ],
)
def _conv_kernel(rowsb_hbm, colp_hbm, m_hbm, zeros_hbm, acc_hbm,
                 rows_v, cols_v, xg_v, acc_sp, sem):
    c = lax.axis_index("c")
    s = lax.axis_index("s")

    for p in range(NPASS):
        for i in range(RSUB // RZCH):
            zo = pl.multiple_of(s * RSUB + i * RZCH, 8)
            pltpu.sync_copy(zeros_hbm, acc_sp.at[pl.ds(zo, RZCH)])
        plsc.subcore_barrier()

        rbase = pl.multiple_of(c * EPD + s * PER_SUB, 8)
        cbase = pl.multiple_of(p * (2 * EPD) + c * EPD + s * PER_SUB, 8)

        def chunk(k, carry):
            roff = pl.multiple_of(rbase + k * KC, 8)
            coff = pl.multiple_of(cbase + k * KC, 8)
            pltpu.sync_copy(rowsb_hbm.at[pl.ds(roff, KC)], rows_v)
            pltpu.sync_copy(colp_hbm.at[pl.ds(coff, KC)], cols_v)
            pltpu.async_copy(m_hbm.at[rows_v], xg_v, sem).wait()
            pltpu.sync_copy(xg_v, acc_sp.at[cols_v], add=True)
            return carry

        lax.fori_loop(0, CCHUNKS, chunk, 0)
        plsc.subcore_barrier()
        obase = c * (NPASS * PROWS) + p * PROWS + s * OSUB
        for i in range(OSUB // OCH):
            so = pl.multiple_of(s * OSUB + i * OCH, 8)
            oo = pl.multiple_of(obase + i * OCH, 8)
            pltpu.sync_copy(acc_sp.at[pl.ds(so, OCH)],
                            acc_hbm.at[pl.ds(oo, OCH)])


@functools.partial(
    pl.kernel, mesh=_mesh,
    out_type=jax.ShapeDtypeStruct((NBG * 2 * BFLAT,), jnp.float32),
    scratch_types=[
        pltpu.VMEM((K,), jnp.int32),
        pltpu.VMEM((K,), jnp.int32),
        pltpu.VMEM((K,), jnp.float32),
        pltpu.VMEM_SHARED((BFLAT,), jnp.float32),
        pltpu.SemaphoreType.DMA,
    ],
)
def _bmat_kernel(rowsb_hbm, flat_hbm, dinv_hbm, zeros_hbm, b_hbm,
                 roff_v, flat_v, dg_v, b_sp, sem):
    c = lax.axis_index("c")
    s = lax.axis_index("s")
    zoff = pl.multiple_of(s * BSUB, 8)

    for g in range(NBG):
        for i in range(BSUB // BCH):
            pltpu.sync_copy(zeros_hbm.at[pl.ds(0, BCH)],
                            b_sp.at[pl.ds(zoff + i * BCH, BCH)])
        plsc.subcore_barrier()

        base = pl.multiple_of(g * (2 * EPD) + c * EPD + s * PER_SUB, 8)

        def chunk(k, carry):
            off = pl.multiple_of(base + k * K, 8)
            pltpu.sync_copy(rowsb_hbm.at[pl.ds(off, K)], roff_v)
            pltpu.sync_copy(flat_hbm.at[pl.ds(off, K)], flat_v)
            pltpu.async_copy(dinv_hbm.at[roff_v], dg_v, sem).wait()
            pltpu.sync_copy(dg_v, b_sp.at[flat_v], add=True)
            return carry

        lax.fori_loop(0, CHUNKS, chunk, 0)
        plsc.subcore_barrier()
        for i in range(BSUB // BCH):
            oo = pl.multiple_of(g * (2 * BFLAT) + c * BFLAT
                                + zoff + i * BCH, 8)
            pltpu.sync_copy(b_sp.at[pl.ds(zoff + i * BCH, BCH)],
                            b_hbm.at[pl.ds(oo, BCH)])


@functools.partial(
    pl.kernel, mesh=_mesh,
    out_type=[jax.ShapeDtypeStruct((B, DO), jnp.float32),
              jax.ShapeDtypeStruct((B, DO), jnp.float32)],
    scratch_types=[
        pltpu.VMEM((B // 32,), jnp.int32),
        pltpu.VMEM((B // 32, DO), jnp.float32),
        pltpu.SemaphoreType.DMA,
    ],
)
def _gather_kernel(x_hbm, r_hbm, sub_hbm, rel_hbm, sub_out, rel_out,
                   idx_v, rows_v, sem):
    c = lax.axis_index("c")
    s = lax.axis_index("s")
    nb = B // 32
    base = pl.multiple_of((s * 2 + c) * nb, 8)
    pltpu.sync_copy(sub_hbm.at[pl.ds(base, nb)], idx_v)
    pltpu.async_copy(x_hbm.at[idx_v], rows_v, sem).wait()
    pltpu.sync_copy(rows_v, sub_out.at[pl.ds(base, nb)])
    pltpu.sync_copy(rel_hbm.at[pl.ds(base, nb)], idx_v)
    pltpu.async_copy(r_hbm.at[idx_v], rows_v, sem).wait()
    pltpu.sync_copy(rows_v, rel_out.at[pl.ds(base, nb)])


_R = 2000


def _init_body(feat_ref, id_ref, g_ref, a_ref, l_ref, out_ref):
    feat = feat_ref[...]
    iota = lax.broadcasted_iota(jnp.int32, (_R, 16), 1)

    def lut(tab, col):
        oh = (feat[:, col:col + 1] == iota).astype(jnp.float32)
        return jnp.dot(oh, tab, preferred_element_type=jnp.float32)

    out_ref[...] = jnp.concatenate(
        [id_ref[...], lut(g_ref[...], 0), lut(a_ref[...], 1),
         lut(l_ref[...], 2)], axis=1)


def _m_body(degi_ref, dego_ref, x_ref, wi_ref, wo_ref,
            dvi_ref, dvo_ref, mi_ref, mo_ref):
    x = x_ref[...]
    di = degi_ref[...]
    do = dego_ref[...]
    dvi = jnp.where(di > 0, lax.rsqrt(di), 0.0)
    dvo = jnp.where(do > 0, lax.rsqrt(do), 0.0)
    dvi_ref[...] = dvi
    dvo_ref[...] = dvo
    mi_ref[...] = jnp.dot(x * dvi, wi_ref[...],
                          preferred_element_type=jnp.float32)
    mo_ref[...] = jnp.dot(x * dvo, wo_ref[...],
                          preferred_element_type=jnp.float32)


def _final_body(ai_ref, ao_ref, bi_ref, bo_ref, dvi_ref, dvo_ref, x_ref,
                lr_ref, ri_ref, ro_ref, wi_ref, wo_ref, wl_ref, b_ref,
                ir_ref, wr_ref, xo_ref, rlo_ref):
    rw_i = jnp.dot(ri_ref[...], wi_ref[...],
                   preferred_element_type=jnp.float32)
    rw_o = jnp.dot(ro_ref[...], wo_ref[...],
                   preferred_element_type=jnp.float32)
    h_in = (ai_ref[...] - jnp.dot(bi_ref[...], rw_i,
                                  preferred_element_type=jnp.float32)
            ) * dvi_ref[...]
    h_out = (ao_ref[...] - jnp.dot(bo_ref[...], rw_o,
                                   preferred_element_type=jnp.float32)
             ) * dvo_ref[...]
    hsum = h_in + h_out + jnp.dot(x_ref[...] - lr_ref[...], wl_ref[...],
                                  preferred_element_type=jnp.float32)
    xo_ref[...] = jnp.tanh(hsum * (1.0 / 3.0) + b_ref[...])
    rlo_ref[...] = jnp.dot(ir_ref[...], wr_ref[...],
                           preferred_element_type=jnp.float32)


def _row_spec(cols):
    return pl.BlockSpec((_R, cols), lambda i: (i, 0))


def _full_spec(r, cols):
    return pl.BlockSpec((r, cols), lambda i: (0, 0))


def kernel(sub, rel, edge_index, edge_type, ent_feature, id_embed,
           gender_table, age_table, level_table, init_rel, loop_rel,
           w_in, w_out, w_loop, w_rel, b_conv):
    i32 = jnp.int32
    f32 = jnp.float32
    sub = sub.astype(i32)
    rel = rel.astype(i32)
    rows = edge_index[0].astype(i32)
    cols = edge_index[1].astype(i32)
    et = edge_type.astype(i32)

    spread = jnp.tile(jnp.arange(1200, dtype=i32), 8)
    pad_hi = N + spread
    rows_deg = jnp.concatenate([rows[:H], pad_hi, rows[H:], pad_hi])
    cols_m = jnp.concatenate([cols[:H], pad_hi, cols[H:], pad_hi])
    r0 = jnp.concatenate([rows[:H], spread])
    r1 = jnp.concatenate([rows[H:], spread])
    rowsb = jnp.concatenate([r0, r1 + N])
    colps = []
    for p in range(NPASS):
        lo = p * PROWS
        cshift = cols_m - lo
        colps.append(jnp.where((cshift >= 0) & (cshift < PROWS),
                               cshift, TRASH))
    colp_all = jnp.concatenate(colps)
    et2 = jnp.concatenate([et[:H], jnp.zeros((PAD,), i32),
                           et[H:] - NT, jnp.zeros((PAD,), i32)])
    rowsb_gs = []
    flat_gs = []
    for g in range(NBG):
        ing = (et2 >= g * NTG) & (et2 < (g + 1) * NTG)
        rowsb_gs.append(jnp.where(ing, rowsb, 0))
        flat_gs.append(jnp.where(ing, cols_m * NTG + (et2 - g * NTG),
                                 BTRASH))
    rowsb_g = jnp.concatenate(rowsb_gs)
    flat_g = jnp.concatenate(flat_gs)

    featp = jnp.pad(ent_feature.astype(i32), ((0, 0), (0, 5)))
    gpad = jnp.pad(gender_table, ((0, 13), (0, 0)))
    apad = jnp.pad(age_table, ((0, 7), (0, 0)))
    lpad = jnp.pad(level_table, ((0, 5), (0, 0)))
    x0 = pl.pallas_call(
        _init_body,
        grid=(N // _R,),
        in_specs=[pl.BlockSpec((_R, 8), lambda i: (i, 0)),
                  _row_spec(16), _full_spec(16, 16), _full_spec(16, 16),
                  _full_spec(16, 16)],
        out_specs=_row_spec(D),
        out_shape=jax.ShapeDtypeStruct((N, D), f32),
    )(featp, id_embed, gpad, apad, lpad)

    zeros1 = jnp.zeros((ZROWS,), f32)
    deg_flat = _deg_kernel(rows_deg, zeros1)
    deg_i = deg_flat[:N].reshape(N, 1)
    deg_o = deg_flat[NPAD:NPAD + N].reshape(N, 1)

    dvi, dvo, m_i, m_o = pl.pallas_call(
        _m_body,
        grid=(N // _R,),
        in_specs=[_row_spec(1), _row_spec(1), _row_spec(D),
                  _full_spec(D, DO), _full_spec(D, DO)],
        out_specs=[_row_spec(1), _row_spec(1), _row_spec(DO), _row_spec(DO)],
        out_shape=[jax.ShapeDtypeStruct((N, 1), f32),
                   jax.ShapeDtypeStruct((N, 1), f32),
                   jax.ShapeDtypeStruct((N, DO), f32),
                   jax.ShapeDtypeStruct((N, DO), f32)],
    )(deg_i, deg_o, x0, w_in, w_out)
    m_stack = jnp.concatenate([m_i, m_o], axis=0)

    zeros2 = jnp.zeros((RZCH, DO), f32)
    acc_flat = _conv_kernel(rowsb, colp_all, m_stack, zeros2)
    a_in = acc_flat[:N]
    a_out = acc_flat[NPASS * PROWS:NPASS * PROWS + N]

    dinv_stack = jnp.concatenate([dvi.reshape(N), dvo.reshape(N)])
    zerosb = jnp.zeros((BCH,), f32)
    b_flat = _bmat_kernel(rowsb_g, flat_g, dinv_stack, zerosb)
    bmat = b_flat.reshape(NBG, 2, NPAD, NTG)
    bi = jnp.concatenate([bmat[g, 0, :N] for g in range(NBG)], axis=1)
    bo = jnp.concatenate([bmat[g, 1, :N] for g in range(NBG)], axis=1)

    bb = b_conv.reshape(1, DO)
    rel_in = init_rel[:NT]
    rel_out = init_rel[NT:]
    x_out, r_out = pl.pallas_call(
        _final_body,
        grid=(N // _R,),
        in_specs=[_row_spec(DO), _row_spec(DO),
                  _row_spec(NT), _row_spec(NT),
                  _row_spec(1), _row_spec(1), _row_spec(D),
                  _full_spec(1, D), _full_spec(NT, D), _full_spec(NT, D),
                  _full_spec(D, DO), _full_spec(D, DO),
                  _full_spec(D, DO), _full_spec(1, DO), _full_spec(40, D),
                  _full_spec(D, DO)],
        out_specs=[_row_spec(DO), _full_spec(40, DO)],
        out_shape=[jax.ShapeDtypeStruct((N, DO), f32),
                   jax.ShapeDtypeStruct((40, DO), f32)],
    )(a_in, a_out, bi, bo, dvi, dvo, x0, loop_rel, rel_in, rel_out,
      w_in, w_out, w_loop, bb, init_rel, w_rel)

    sub_emb, rel_emb = _gather_kernel(x_out, r_out, sub, rel)
    return (sub_emb, rel_emb, x_out)


# speedup vs baseline: 1.1337x; 1.0587x over previous
import functools

import jax
import jax.numpy as jnp
from jax import lax
from jax.experimental import pallas as pl
from jax.experimental.pallas import tpu as pltpu
from jax.experimental.pallas import tpu_sc as plsc

N = 50000
NREL = 41
NT = 20
E = 800000
H = 400000
D = 64
DO = 128
B = 4096

EPD = 409600
PAD = EPD - H
PER_SUB = 25600
K = 1024
CHUNKS = PER_SUB // K
KC = 320
CCHUNKS = PER_SUB // KC
NPAD = 51200
ZROWS = 3200
NTG = 5
NBG = NT // NTG
BFLAT = NPAD * NTG
BSUB = BFLAT // 16
BCH = 3200
BTRASH = BFLAT - 1
NPASS = 5
PROWS = 10240
RR = 10368
RSUB = 648
RZCH = 216
OSUB = 640
OCH = 320
TRASH = 10240

_mesh = plsc.VectorSubcoreMesh(core_axis_name="c", subcore_axis_name="s")


@functools.partial(
    pl.kernel, mesh=_mesh,
    out_type=jax.ShapeDtypeStruct((2 * NPAD,), jnp.float32),
    scratch_types=[
        pltpu.VMEM((K,), jnp.int32),
        pltpu.VMEM((K,), jnp.float32),
        pltpu.VMEM_SHARED((NPAD,), jnp.float32),
    ],
)
def _deg_kernel(rows_hbm, zeros_hbm, deg_hbm, idx_v, ones_v, deg_sp):
    c = lax.axis_index("c")
    s = lax.axis_index("s")
    zoff = pl.multiple_of(s * ZROWS, 8)
    pltpu.sync_copy(zeros_hbm.at[pl.ds(0, ZROWS)], deg_sp.at[pl.ds(zoff, ZROWS)])
    for i in range(K // 16):
        ones_v[pl.ds(i * 16, 16)] = jnp.full((16,), 1.0, jnp.float32)
    plsc.subcore_barrier()

    base = pl.multiple_of(c * EPD + s * PER_SUB, 8)

    def chunk(k, carry):
        off = pl.multiple_of(base + k * K, 8)
        pltpu.sync_copy(rows_hbm.at[pl.ds(off, K)], idx_v)
        pltpu.sync_copy(ones_v, deg_sp.at[idx_v], add=True)
        return carry

    lax.fori_loop(0, CHUNKS, chunk, 0)
    plsc.subcore_barrier()
    ooff = pl.multiple_of(c * NPAD + s * ZROWS, 8)
    pltpu.sync_copy(deg_sp.at[pl.ds(zoff, ZROWS)], deg_hbm.at[pl.ds(ooff, ZROWS)])


@functools.partial(
    pl.kernel, mesh=_mesh,
    out_type=jax.ShapeDtypeStruct((2 * NPASS * PROWS, DO), jnp.float32),
    scratch_types=[
        pltpu.VMEM((KC,), jnp.int32),
        pltpu.VMEM((KC,), jnp.int32),
        pltpu.VMEM((KC, DO), jnp.float32),
        pltpu.VMEM_SHARED((RR, DO), jnp.float32),
        pltpu.SemaphoreType.DMA,
    ],
)
def _conv_kernel(rowsb_hbm, colp_hbm, m_hbm, zeros_hbm, acc_hbm,
                 rows_v, cols_v, xg_v, acc_sp, sem):
    c = lax.axis_index("c")
    s = lax.axis_index("s")

    for p in range(NPASS):
        for i in range(RSUB // RZCH):
            zo = pl.multiple_of(s * RSUB + i * RZCH, 8)
            pltpu.sync_copy(zeros_hbm, acc_sp.at[pl.ds(zo, RZCH)])
        plsc.subcore_barrier()

        rbase = pl.multiple_of(c * EPD + s * PER_SUB, 8)
        cbase = pl.multiple_of(p * (2 * EPD) + c * EPD + s * PER_SUB, 8)

        def chunk(k, carry):
            roff = pl.multiple_of(rbase + k * KC, 8)
            coff = pl.multiple_of(cbase + k * KC, 8)
            pltpu.sync_copy(rowsb_hbm.at[pl.ds(roff, KC)], rows_v)
            pltpu.sync_copy(colp_hbm.at[pl.ds(coff, KC)], cols_v)
            pltpu.async_copy(m_hbm.at[rows_v], xg_v, sem).wait()
            pltpu.sync_copy(xg_v, acc_sp.at[cols_v], add=True)
            return carry

        lax.fori_loop(0, CCHUNKS, chunk, 0)
        plsc.subcore_barrier()
        obase = c * (NPASS * PROWS) + p * PROWS + s * OSUB
        for i in range(OSUB // OCH):
            so = pl.multiple_of(s * OSUB + i * OCH, 8)
            oo = pl.multiple_of(obase + i * OCH, 8)
            pltpu.sync_copy(acc_sp.at[pl.ds(so, OCH)],
                            acc_hbm.at[pl.ds(oo, OCH)])


@functools.partial(
    pl.kernel, mesh=_mesh,
    out_type=jax.ShapeDtypeStruct((NBG * 2 * BFLAT,), jnp.float32),
    scratch_types=[
        pltpu.VMEM((K,), jnp.int32),
        pltpu.VMEM((K,), jnp.int32),
        pltpu.VMEM((K,), jnp.float32),
        pltpu.VMEM_SHARED((BFLAT,), jnp.float32),
        pltpu.SemaphoreType.DMA,
    ],
)
def _bmat_kernel(rowsb_hbm, flat_hbm, dinv_hbm, zeros_hbm, b_hbm,
                 roff_v, flat_v, dg_v, b_sp, sem):
    c = lax.axis_index("c")
    s = lax.axis_index("s")
    zoff = pl.multiple_of(s * BSUB, 8)

    for g in range(NBG):
        for i in range(BSUB // BCH):
            pltpu.sync_copy(zeros_hbm.at[pl.ds(0, BCH)],
                            b_sp.at[pl.ds(zoff + i * BCH, BCH)])
        plsc.subcore_barrier()

        base = pl.multiple_of(g * (2 * EPD) + c * EPD + s * PER_SUB, 8)

        def chunk(k, carry):
            off = pl.multiple_of(base + k * K, 8)
            pltpu.sync_copy(rowsb_hbm.at[pl.ds(off, K)], roff_v)
            pltpu.sync_copy(flat_hbm.at[pl.ds(off, K)], flat_v)
            pltpu.async_copy(dinv_hbm.at[roff_v], dg_v, sem).wait()
            pltpu.sync_copy(dg_v, b_sp.at[flat_v], add=True)
            return carry

        lax.fori_loop(0, CHUNKS, chunk, 0)
        plsc.subcore_barrier()
        for i in range(BSUB // BCH):
            oo = pl.multiple_of(g * (2 * BFLAT) + c * BFLAT
                                + zoff + i * BCH, 8)
            pltpu.sync_copy(b_sp.at[pl.ds(zoff + i * BCH, BCH)],
                            b_hbm.at[pl.ds(oo, BCH)])


@functools.partial(
    pl.kernel, mesh=_mesh,
    out_type=[jax.ShapeDtypeStruct((B, DO), jnp.float32),
              jax.ShapeDtypeStruct((B, DO), jnp.float32)],
    scratch_types=[
        pltpu.VMEM((B // 32,), jnp.int32),
        pltpu.VMEM((B // 32, DO), jnp.float32),
        pltpu.SemaphoreType.DMA,
    ],
)
def _gather_kernel(x_hbm, r_hbm, sub_hbm, rel_hbm, sub_out, rel_out,
                   idx_v, rows_v, sem):
    c = lax.axis_index("c")
    s = lax.axis_index("s")
    nb = B // 32
    base = pl.multiple_of((s * 2 + c) * nb, 8)
    pltpu.sync_copy(sub_hbm.at[pl.ds(base, nb)], idx_v)
    pltpu.async_copy(x_hbm.at[idx_v], rows_v, sem).wait()
    pltpu.sync_copy(rows_v, sub_out.at[pl.ds(base, nb)])
    pltpu.sync_copy(rel_hbm.at[pl.ds(base, nb)], idx_v)
    pltpu.async_copy(r_hbm.at[idx_v], rows_v, sem).wait()
    pltpu.sync_copy(rows_v, rel_out.at[pl.ds(base, nb)])


_R = 2000


def _init_body(feat_ref, id_ref, g_ref, a_ref, l_ref, out_ref):
    feat = feat_ref[...]
    iota = lax.broadcasted_iota(jnp.int32, (_R, 16), 1)

    def lut(tab, col):
        oh = (feat[:, col:col + 1] == iota).astype(jnp.float32)
        return jnp.dot(oh, tab, preferred_element_type=jnp.float32)

    out_ref[...] = jnp.concatenate(
        [id_ref[...], lut(g_ref[...], 0), lut(a_ref[...], 1),
         lut(l_ref[...], 2)], axis=1)


def _m_body(degi_ref, dego_ref, x_ref, wi_ref, wo_ref,
            dvi_ref, dvo_ref, mi_ref, mo_ref):
    x = x_ref[...]
    di = degi_ref[...]
    do = dego_ref[...]
    dvi = jnp.where(di > 0, lax.rsqrt(di), 0.0)
    dvo = jnp.where(do > 0, lax.rsqrt(do), 0.0)
    dvi_ref[...] = dvi
    dvo_ref[...] = dvo
    mi_ref[...] = jnp.dot(x * dvi, wi_ref[...],
                          preferred_element_type=jnp.float32)
    mo_ref[...] = jnp.dot(x * dvo, wo_ref[...],
                          preferred_element_type=jnp.float32)


def _final_body(ai_ref, ao_ref, bi_ref, bo_ref, dvi_ref, dvo_ref, x_ref,
                lr_ref, ri_ref, ro_ref, wi_ref, wo_ref, wl_ref, b_ref,
                ir_ref, wr_ref, xo_ref, rlo_ref):
    rw_i = jnp.dot(ri_ref[...], wi_ref[...],
                   preferred_element_type=jnp.float32)
    rw_o = jnp.dot(ro_ref[...], wo_ref[...],
                   preferred_element_type=jnp.float32)
    h_in = (ai_ref[...] - jnp.dot(bi_ref[...], rw_i,
                                  preferred_element_type=jnp.float32)
            ) * dvi_ref[...]
    h_out = (ao_ref[...] - jnp.dot(bo_ref[...], rw_o,
                                   preferred_element_type=jnp.float32)
             ) * dvo_ref[...]
    hsum = h_in + h_out + jnp.dot(x_ref[...] - lr_ref[...], wl_ref[...],
                                  preferred_element_type=jnp.float32)
    xo_ref[...] = jnp.tanh(hsum * (1.0 / 3.0) + b_ref[...])
    rlo_ref[...] = jnp.dot(ir_ref[...], wr_ref[...],
                           preferred_element_type=jnp.float32)


def _row_spec(cols):
    return pl.BlockSpec((_R, cols), lambda i: (i, 0))


def _full_spec(r, cols):
    return pl.BlockSpec((r, cols), lambda i: (0, 0))


def kernel(sub, rel, edge_index, edge_type, ent_feature, id_embed,
           gender_table, age_table, level_table, init_rel, loop_rel,
           w_in, w_out, w_loop, w_rel, b_conv):
    i32 = jnp.int32
    f32 = jnp.float32
    sub = sub.astype(i32)
    rel = rel.astype(i32)
    rows = edge_index[0].astype(i32)
    cols = edge_index[1].astype(i32)
    et = edge_type.astype(i32)

    spread = jnp.tile(jnp.arange(1200, dtype=i32), 8)
    pad_hi = N + spread
    rows_deg = jnp.concatenate([rows[:H], pad_hi, rows[H:], pad_hi])
    cols_m = jnp.concatenate([cols[:H], pad_hi, cols[H:], pad_hi])
    r0 = jnp.concatenate([rows[:H], spread])
    r1 = jnp.concatenate([rows[H:], spread])
    rowsb = jnp.concatenate([r0, r1 + N])
    colps = []
    for p in range(NPASS):
        lo = p * PROWS
        cshift = cols_m - lo
        colps.append(jnp.where((cshift >= 0) & (cshift < PROWS),
                               cshift, TRASH))
    colp_all = jnp.concatenate(colps)
    et2 = jnp.concatenate([et[:H], jnp.zeros((PAD,), i32),
                           et[H:] - NT, jnp.zeros((PAD,), i32)])
    rowsb_gs = []
    flat_gs = []
    for g in range(NBG):
        ing = (et2 >= g * NTG) & (et2 < (g + 1) * NTG)
        rowsb_gs.append(jnp.where(ing, rowsb, 0))
        flat_gs.append(jnp.where(ing, cols_m * NTG + (et2 - g * NTG),
                                 BTRASH))
    rowsb_g = jnp.concatenate(rowsb_gs)
    flat_g = jnp.concatenate(flat_gs)

    featp = jnp.pad(ent_feature.astype(i32), ((0, 0), (0, 5)))
    gpad = jnp.pad(gender_table, ((0, 13), (0, 0)))
    apad = jnp.pad(age_table, ((0, 7), (0, 0)))
    lpad = jnp.pad(level_table, ((0, 5), (0, 0)))
    x0 = pl.pallas_call(
        _init_body,
        grid=(N // _R,),
        in_specs=[pl.BlockSpec((_R, 8), lambda i: (i, 0)),
                  _row_spec(16), _full_spec(16, 16), _full_spec(16, 16),
                  _full_spec(16, 16)],
        out_specs=_row_spec(D),
        out_shape=jax.ShapeDtypeStruct((N, D), f32),
    )(featp, id_embed, gpad, apad, lpad)

    zeros1 = jnp.zeros((ZROWS,), f32)
    deg_flat = _deg_kernel(rows_deg, zeros1)
    deg_i = deg_flat[:N].reshape(N, 1)
    deg_o = deg_flat[NPAD:NPAD + N].reshape(N, 1)

    dvi, dvo, m_i, m_o = pl.pallas_call(
        _m_body,
        grid=(N // _R,),
        in_specs=[_row_spec(1), _row_spec(1), _row_spec(D),
                  _full_spec(D, DO), _full_spec(D, DO)],
        out_specs=[_row_spec(1), _row_spec(1), _row_spec(DO), _row_spec(DO)],
        out_shape=[jax.ShapeDtypeStruct((N, 1), f32),
                   jax.ShapeDtypeStruct((N, 1), f32),
                   jax.ShapeDtypeStruct((N, DO), f32),
                   jax.ShapeDtypeStruct((N, DO), f32)],
    )(deg_i, deg_o, x0, w_in, w_out)
    m_stack = jnp.concatenate([m_i, m_o], axis=0)

    zeros2 = jnp.zeros((RZCH, DO), f32)
    acc_flat = _conv_kernel(rowsb, colp_all, m_stack, zeros2)
    a_in = acc_flat[:N]
    a_out = acc_flat[NPASS * PROWS:NPASS * PROWS + N]

    dinv_stack = jnp.concatenate([dvi.reshape(N), dvo.reshape(N)])
    zerosb = jnp.zeros((BCH,), f32)
    b_flat = _bmat_kernel(rowsb_g, flat_g, dinv_stack, zerosb)
    bmat = b_flat.reshape(NBG, 2, NPAD, NTG)
    bi = jnp.concatenate([bmat[g, 0, :N] for g in range(NBG)], axis=1)
    bo = jnp.concatenate([bmat[g, 1, :N] for g in range(NBG)], axis=1)

    bb = b_conv.reshape(1, DO)
    rel_in = init_rel[:NT]
    rel_out = init_rel[NT:]
    x_out, r_out = pl.pallas_call(
        _final_body,
        grid=(N // _R,),
        in_specs=[_row_spec(DO), _row_spec(DO),
                  _row_spec(NT), _row_spec(NT),
                  _row_spec(1), _row_spec(1), _row_spec(D),
                  _full_spec(1, D), _full_spec(NT, D), _full_spec(NT, D),
                  _full_spec(D, DO), _full_spec(D, DO),
                  _full_spec(D, DO), _full_spec(1, DO), _full_spec(40, D),
                  _full_spec(D, DO)],
        out_specs=[_row_spec(DO), _full_spec(40, DO)],
        out_shape=[jax.ShapeDtypeStruct((N, DO), f32),
                   jax.ShapeDtypeStruct((40, DO), f32)],
    )(a_in, a_out, bi, bo, dvi, dvo, x0, loop_rel, rel_in, rel_out,
      w_in, w_out, w_loop, bb, init_rel, w_rel)

    sub_emb, rel_emb = _gather_kernel(x_out, r_out, sub, rel)
    return (sub_emb, rel_emb, x_out)
